# baseline (speedup 1.0000x reference)
import jax
import jax.numpy as jnp
from jax import lax
from jax.experimental import pallas as pl
from jax.experimental.pallas import tpu as pltpu
from jax.experimental.pallas import tpu_sc as plsc

VOCAB = 100000
EMBED_DIM = 64
BATCH = 4096
SEQ = 200

NC = 2
NS = 16
NW = NC * NS
LANES = 16
CPR = EMBED_DIM // LANES

ROWS_PER_W = BATCH // NW
SC_ROWS = 2
GCHUNK = 40
NG = SEQ // GCHUNK
N_ITEMS = ROWS_PER_W // SC_ROWS


def _emb_kernel(ids_hbm, tok_hbm, pos_hbm, out_hbm,
                pos_v, idx_v, rows_a, rows_b, gsem_a, gsem_b,
                osem_a, osem_b):
    wid = lax.axis_index("s") * NC + lax.axis_index("c")
    base = wid * ROWS_PER_W

    pltpu.sync_copy(ids_hbm.at[pl.ds(base, ROWS_PER_W)], idx_v)
    pltpu.sync_copy(pos_hbm.at[pl.ds(0, SEQ)], pos_v)

    def fire_gathers(item, rows_v, sem):
        for h in range(SC_ROWS):
            r = item * SC_ROWS + h
            for j in range(NG):
                pltpu.async_copy(
                    tok_hbm.at[idx_v.at[r, pl.ds(j * GCHUNK, GCHUNK)]],
                    rows_v.at[h, pl.ds(j * GCHUNK, GCHUNK)],
                    sem,
                )

    def wait_gathers(item, rows_v, sem):
        for h in range(SC_ROWS):
            r = item * SC_ROWS + h
            for j in range(NG):
                pltpu.make_async_copy(
                    tok_hbm.at[idx_v.at[r, pl.ds(j * GCHUNK, GCHUNK)]],
                    rows_v.at[h, pl.ds(j * GCHUNK, GCHUNK)],
                    sem,
                ).wait()

    def add_pos(rows_v):
        def row_body(r, _):
            for h in range(SC_ROWS):
                for c in range(CPR):
                    rows_v[h, r, pl.ds(c * LANES, LANES)] = (
                        rows_v[h, r, pl.ds(c * LANES, LANES)]
                        + pos_v[r, pl.ds(c * LANES, LANES)]
                    )
            return 0
        lax.fori_loop(0, SEQ, row_body, 0, unroll=2)

    def fire_store(item, rows_v, sem):
        pltpu.async_copy(
            rows_v, out_hbm.at[pl.ds(base + item * SC_ROWS, SC_ROWS)], sem)

    def wait_store(item, rows_v, sem):
        pltpu.make_async_copy(
            rows_v, out_hbm.at[pl.ds(base + item * SC_ROWS, SC_ROWS)], sem,
        ).wait()

    fire_gathers(0, rows_a, gsem_a)
    fire_gathers(1, rows_b, gsem_b)

    def body(g, _):
        ia = 2 * g
        ib = 2 * g + 1
        wait_gathers(ia, rows_a, gsem_a)
        add_pos(rows_a)
        fire_store(ia, rows_a, osem_a)
        wait_gathers(ib, rows_b, gsem_b)
        add_pos(rows_b)
        fire_store(ib, rows_b, osem_b)

        @pl.when(g < N_ITEMS // 2 - 1)
        def _refill():
            wait_store(ia, rows_a, osem_a)
            fire_gathers(ia + 2, rows_a, gsem_a)
            wait_store(ib, rows_b, osem_b)
            fire_gathers(ib + 2, rows_b, gsem_b)

        return 0

    lax.fori_loop(0, N_ITEMS // 2, body, 0)

    wait_store(N_ITEMS - 2, rows_a, osem_a)
    wait_store(N_ITEMS - 1, rows_b, osem_b)


@jax.jit
def _run(input_ids, token_embedding, position_embedding):
    mesh = plsc.VectorSubcoreMesh(core_axis_name="c", subcore_axis_name="s")
    call = pl.kernel(
        _emb_kernel,
        out_type=jax.ShapeDtypeStruct((BATCH, SEQ, EMBED_DIM), jnp.float32),
        mesh=mesh,
        scratch_types=[
            pltpu.VMEM((SEQ, EMBED_DIM), jnp.float32),
            pltpu.VMEM((ROWS_PER_W, SEQ), jnp.int32),
            pltpu.VMEM((SC_ROWS, SEQ, EMBED_DIM), jnp.float32),
            pltpu.VMEM((SC_ROWS, SEQ, EMBED_DIM), jnp.float32),
            pltpu.SemaphoreType.DMA,
            pltpu.SemaphoreType.DMA,
            pltpu.SemaphoreType.DMA,
            pltpu.SemaphoreType.DMA,
        ],
        compiler_params=pltpu.CompilerParams(use_tc_tiling_on_sc=False),
    )
    return call(input_ids, token_embedding, position_embedding)


def kernel(input_ids, token_embedding, position_embedding):
    return _run(input_ids.astype(jnp.int32), token_embedding,
                position_embedding)


# speedup vs baseline: 1.5162x; 1.5162x over previous
import jax
import jax.numpy as jnp
from jax import lax
from jax.experimental import pallas as pl
from jax.experimental.pallas import tpu as pltpu
from jax.experimental.pallas import tpu_sc as plsc

VOCAB = 100000
EMBED_DIM = 64
BATCH = 4096
SEQ = 200

NC = 2
NS = 16
NW = NC * NS
LANES = 16
CPR = EMBED_DIM // LANES

ROWS_PER_W = BATCH // NW
GCHUNK = 40
NG = SEQ // GCHUNK
NBUF = 4
AHEAD = 2


def _emb_kernel(ids_hbm, tok_hbm, pos_hbm, out_hbm,
                pos_v, idx_v, rows, gsems, osems):
    wid = lax.axis_index("s") * NC + lax.axis_index("c")
    base = wid * ROWS_PER_W

    pltpu.sync_copy(ids_hbm.at[pl.ds(base, ROWS_PER_W)], idx_v)
    pltpu.sync_copy(pos_hbm.at[pl.ds(0, SEQ)], pos_v)

    def fire_gathers(item, k):
        for j in range(NG):
            pltpu.async_copy(
                tok_hbm.at[idx_v.at[item, pl.ds(j * GCHUNK, GCHUNK)]],
                rows[k].at[pl.ds(j * GCHUNK, GCHUNK)],
                gsems[k],
            )

    def wait_gathers(item, k):
        for j in range(NG):
            pltpu.make_async_copy(
                tok_hbm.at[idx_v.at[item, pl.ds(j * GCHUNK, GCHUNK)]],
                rows[k].at[pl.ds(j * GCHUNK, GCHUNK)],
                gsems[k],
            ).wait()

    def add_pos(k):
        rows_v = rows[k]

        def row_body(r):
            for c in range(CPR):
                rows_v[r, pl.ds(c * LANES, LANES)] = (
                    rows_v[r, pl.ds(c * LANES, LANES)]
                    + pos_v[r, pl.ds(c * LANES, LANES)]
                )

        plsc.parallel_loop(0, SEQ, unroll=4)(row_body)

    def fire_store(item, k):
        pltpu.async_copy(rows[k], out_hbm.at[base + item], osems[k])

    def wait_store(item, k):
        pltpu.make_async_copy(
            rows[k], out_hbm.at[base + item], osems[k]).wait()

    fire_gathers(0, 0)
    fire_gathers(1, 1)

    def body(gg, _):
        for k in range(NBUF):
            i = NBUF * gg + k
            wait_gathers(i, k)
            add_pos(k)
            fire_store(i, k)
            j = i + AHEAD
            kj = (k + AHEAD) % NBUF

            @pl.when(j < ROWS_PER_W)
            def _fire_ahead():
                @pl.when(j >= NBUF)
                def _drain():
                    wait_store(j - NBUF, kj)
                fire_gathers(j, kj)

        return 0

    lax.fori_loop(0, ROWS_PER_W // NBUF, body, 0)

    for k in range(NBUF):
        wait_store(ROWS_PER_W - NBUF + k, k)


@jax.jit
def _run(input_ids, token_embedding, position_embedding):
    mesh = plsc.VectorSubcoreMesh(core_axis_name="c", subcore_axis_name="s")

    def entry(ids_hbm, tok_hbm, pos_hbm, out_hbm,
              pos_v, idx_v, r0, r1, r2, r3, g0, g1, g2, g3,
              o0, o1, o2, o3):
        _emb_kernel(ids_hbm, tok_hbm, pos_hbm, out_hbm, pos_v, idx_v,
                    [r0, r1, r2, r3], [g0, g1, g2, g3],
                    [o0, o1, o2, o3])

    call = pl.kernel(
        entry,
        out_type=jax.ShapeDtypeStruct((BATCH, SEQ, EMBED_DIM), jnp.float32),
        mesh=mesh,
        scratch_types=(
            [pltpu.VMEM((SEQ, EMBED_DIM), jnp.float32),
             pltpu.VMEM((ROWS_PER_W, SEQ), jnp.int32)]
            + [pltpu.VMEM((SEQ, EMBED_DIM), jnp.float32)
               for _ in range(NBUF)]
            + [pltpu.SemaphoreType.DMA for _ in range(2 * NBUF)]
        ),
        compiler_params=pltpu.CompilerParams(use_tc_tiling_on_sc=False),
    )
    return call(input_ids, token_embedding, position_embedding)


def kernel(input_ids, token_embedding, position_embedding):
    return _run(input_ids.astype(jnp.int32), token_embedding,
                position_embedding)


# speedup vs baseline: 4.9282x; 3.2504x over previous
import jax
import jax.numpy as jnp
from jax import lax
from jax.experimental import pallas as pl
from jax.experimental.pallas import tpu as pltpu
from jax.experimental.pallas import tpu_sc as plsc

VOCAB = 100000
EMBED_DIM = 64
BATCH = 4096
SEQ = 200

NC = 2
NS = 16
NW = NC * NS
LANES = 16
CPR = EMBED_DIM // LANES

BBLK = BATCH // NW
DT = EMBED_DIM // 8
NBUF = 5
AHEAD = 4


def _emb_kernel(ids_hbm, tok_hbm, pos_hbm, out_hbm,
                pos_v, idx_t, grows, slabs, gsems, osems):
    wid = lax.axis_index("s") * NC + lax.axis_index("c")

    pltpu.sync_copy(pos_hbm.at[pl.ds(0, SEQ)], pos_v)

    pltpu.sync_copy(ids_hbm.at[:, wid], idx_t)

    def fire_gather(s, k):
        pltpu.async_copy(
            tok_hbm.at[idx_t.at[s >> 3, s & 7]], grows[k], gsems[k])

    def wait_gather(s, k):
        pltpu.make_async_copy(
            tok_hbm.at[idx_t.at[s >> 3, s & 7]], grows[k], gsems[k]).wait()

    def transpose_add(s, k):
        grows_v = grows[k]
        slab_v = slabs[k]
        iota16 = lax.iota(jnp.int32, 16)
        rrs = [(16 * c + iota16) >> 3 for c in range(CPR)]
        dis = [(16 * c + iota16) & 7 for c in range(CPR)]
        pvs = [pos_v[s, pl.ds(c * LANES, LANES)] for c in range(CPR)]

        def bi_body(bi):
            bv = jnp.full((16,), bi, jnp.int32)
            for c in range(CPR):
                v = grows_v[bi, pl.ds(c * LANES, LANES)] + pvs[c]
                plsc.store_scatter(slab_v, [rrs[c], dis[c], bv], v)

        plsc.parallel_loop(0, BBLK, unroll=4)(bi_body)

    def fire_store(s, k):
        pltpu.async_copy(slabs[k].at[:, :, pl.ds(0, BBLK)],
                         out_hbm.at[s, :, wid], osems[k])

    def wait_store(s, k):
        pltpu.make_async_copy(
            slabs[k].at[:, :, pl.ds(0, BBLK)],
            out_hbm.at[s, :, wid], osems[k]).wait()

    for p in range(AHEAD):
        fire_gather(p, p)

    def body(gg, _):
        for k in range(NBUF):
            i = NBUF * gg + k
            wait_gather(i, k)

            @pl.when(i >= NBUF)
            def _drain():
                wait_store(i - NBUF, k)

            transpose_add(i, k)
            fire_store(i, k)
            j = i + AHEAD

            @pl.when(j < SEQ)
            def _ahead():
                fire_gather(j, (k + AHEAD) % NBUF)

        return 0

    lax.fori_loop(0, SEQ // NBUF, body, 0)

    for k in range(NBUF):
        wait_store(SEQ - NBUF + k, k)


@jax.jit
def _run(input_ids, token_embedding, position_embedding):
    mesh = plsc.VectorSubcoreMesh(core_axis_name="c", subcore_axis_name="s")

    def entry(ids_hbm, tok_hbm, pos_hbm, out_hbm, pos_v, idx_t,
              g0, g1, g2, g3, g4, s0, s1, s2, s3, s4,
              gs0, gs1, gs2, gs3, gs4, os0, os1, os2, os3, os4):
        _emb_kernel(ids_hbm, tok_hbm, pos_hbm, out_hbm, pos_v, idx_t,
                    [g0, g1, g2, g3, g4], [s0, s1, s2, s3, s4],
                    [gs0, gs1, gs2, gs3, gs4], [os0, os1, os2, os3, os4])

    call = pl.kernel(
        entry,
        out_type=jax.ShapeDtypeStruct((SEQ, DT, NW, 8, BBLK), jnp.float32),
        mesh=mesh,
        scratch_types=(
            [pltpu.VMEM((SEQ, EMBED_DIM), jnp.float32),
             pltpu.VMEM((SEQ // 8, 8, BBLK), jnp.int32)]
            + [pltpu.VMEM((BBLK, EMBED_DIM), jnp.float32)
               for _ in range(NBUF)]
            + [pltpu.VMEM((DT, 8, BBLK + 1), jnp.float32)
               for _ in range(NBUF)]
            + [pltpu.SemaphoreType.DMA for _ in range(2 * NBUF)]
        ),
        compiler_params=pltpu.CompilerParams(use_tc_tiling_on_sc=False,
                                             needs_layout_passes=False),
    )
    ids_n = jnp.transpose(
        input_ids.T.reshape(SEQ // 8, 8, NW, BBLK), (0, 2, 1, 3))
    raw = call(ids_n, token_embedding, position_embedding)
    return jnp.transpose(raw, (2, 4, 0, 1, 3)).reshape(BATCH, SEQ, EMBED_DIM)


def kernel(input_ids, token_embedding, position_embedding):
    return _run(input_ids.astype(jnp.int32), token_embedding,
                position_embedding)
